# baseline (speedup 1.0000x reference)
import jax
import jax.numpy as jnp
from jax import lax
from jax.experimental import pallas as pl
from jax.experimental.pallas import tpu as pltpu
from jax.experimental.pallas import tpu_sc as plsc

_N_TILES = 1024
_EMBED = 32
_HID = 256
_NA = 4
_B = 4096

_NC, _NS = 2, 16
_NW = _NC * _NS
_BPW = _B // _NW
_CHUNK = 128
_NCHUNK = _N_TILES // _CHUNK


def _sc_gather_body(table_hbm, board_hbm, out_hbm, idx_v, emb_v, sem):
    w = lax.axis_index("s") * _NC + lax.axis_index("c")
    base = w * _BPW

    def row_body(i, carry):
        br = base + i
        pltpu.sync_copy(board_hbm.at[br], idx_v)
        copies = []
        for j in range(_NCHUNK):
            copies.append(pltpu.async_copy(
                table_hbm.at[idx_v.at[j]],
                emb_v.at[pl.ds(j * _CHUNK, _CHUNK), :],
                sem,
            ))
        for c in copies:
            c.wait()
        pltpu.sync_copy(emb_v, out_hbm.at[br])
        return carry

    lax.fori_loop(0, _BPW, row_body, 0)


def _sc_gather(emb_table, board3):
    mesh = plsc.VectorSubcoreMesh(core_axis_name="c", subcore_axis_name="s")
    f = pl.kernel(
        _sc_gather_body,
        out_type=jax.ShapeDtypeStruct((_B, _N_TILES, _EMBED), jnp.float32),
        mesh=mesh,
        scratch_types=[
            pltpu.VMEM((_NCHUNK, _CHUNK), jnp.int32),
            pltpu.VMEM((_N_TILES, _EMBED), jnp.float32),
            pltpu.SemaphoreType.DMA,
        ],
        compiler_params=pltpu.CompilerParams(use_tc_tiling_on_sc=False),
    )
    return f(emb_table, board3)


_BB = 512
_KB = 4096
_K = _N_TILES * _EMBED


def _mlp_body(flat_ref, w1_ref, b1_ref, w2_ref, b2_ref, w3_ref, b3_ref,
              out_ref, acc_ref):
    k = pl.program_id(0)
    b = pl.program_id(1)
    nk = pl.num_programs(0)
    x = flat_ref[...].reshape(_BB, _KB)
    part = jnp.dot(x, w1_ref[...], preferred_element_type=jnp.float32)
    sl = pl.ds(b * _BB, _BB)

    @pl.when(k == 0)
    def _():
        acc_ref[sl, :] = part

    @pl.when(k > 0)
    def _():
        acc_ref[sl, :] = acc_ref[sl, :] + part

    @pl.when(k == nk - 1)
    def _():
        h1 = jnp.maximum(acc_ref[sl, :] + b1_ref[...], 0.0)
        h2 = jnp.dot(h1, w2_ref[...], preferred_element_type=jnp.float32)
        h2 = jnp.maximum(h2 + b2_ref[...], 0.0)
        out_ref[...] = (
            jnp.dot(h2, w3_ref[...], preferred_element_type=jnp.float32)
            + b3_ref[...]
        )


def _tc_mlp(flat3, W1, b1, W2, b2, W3, b3):
    grid = (_K // _KB, _B // _BB)
    return pl.pallas_call(
        _mlp_body,
        grid=grid,
        in_specs=[
            pl.BlockSpec((_BB, _KB // 128, 128), lambda k, b: (b, k, 0)),
            pl.BlockSpec((_KB, _HID), lambda k, b: (k, 0)),
            pl.BlockSpec((1, _HID), lambda k, b: (0, 0)),
            pl.BlockSpec((_HID, _HID), lambda k, b: (0, 0)),
            pl.BlockSpec((1, _HID), lambda k, b: (0, 0)),
            pl.BlockSpec((_HID, _NA), lambda k, b: (0, 0)),
            pl.BlockSpec((1, _NA), lambda k, b: (0, 0)),
        ],
        out_specs=pl.BlockSpec((_BB, _NA), lambda k, b: (b, 0)),
        out_shape=jax.ShapeDtypeStruct((_B, _NA), jnp.float32),
        scratch_shapes=[pltpu.VMEM((_B, _HID), jnp.float32)],
        compiler_params=pltpu.CompilerParams(
            dimension_semantics=("arbitrary", "arbitrary"),
        ),
    )(flat3, W1, b1, W2, b2, W3, b3)


def kernel(board, emb_table, W1, b1, W2, b2, W3, b3):
    board3 = board.astype(jnp.int32).reshape(_B, _NCHUNK, _CHUNK)
    emb3d = _sc_gather(emb_table, board3)
    flat3 = emb3d.reshape(_B, _K // 128, 128)
    return _tc_mlp(
        flat3, W1, b1.reshape(1, _HID), W2, b2.reshape(1, _HID),
        W3, b3.reshape(1, _NA),
    )


# speedup vs baseline: 2.1593x; 2.1593x over previous
import jax
import jax.numpy as jnp
from jax import lax
from jax.experimental import pallas as pl
from jax.experimental.pallas import tpu as pltpu
from jax.experimental.pallas import tpu_sc as plsc

_N_TILES = 1024
_EMBED = 32
_HID = 256
_NA = 4
_B = 4096

_NC, _NS = 2, 16
_NW = _NC * _NS
_BPW = _B // _NW


def _sc_gather_body(table_hbm, board_hbm, out_hbm, table_v, idx_v, emb_v,
                    sem_out, sem_idx):
    w = lax.axis_index("s") * _NC + lax.axis_index("c")
    base = w * _BPW
    pltpu.sync_copy(table_hbm, table_v)
    iota = lax.iota(jnp.int32, 16)
    c127 = jnp.full((16,), 127, jnp.int32)

    pltpu.async_copy(board_hbm.at[base], idx_v.at[0], sem_idx).wait()

    def row_body(i, carry):
        br = base + i
        p = lax.rem(i, 2)
        @pl.when(i + 1 < _BPW)
        def _():
            pltpu.async_copy(board_hbm.at[br + 1], idx_v.at[1 - p], sem_idx)
        @pl.when(i >= 2)
        def _():
            pltpu.make_async_copy(emb_v.at[p], out_hbm.at[base], sem_out).wait()

        def grp_body(g, carry2):
            n0 = g * 4
            rowv = jnp.full((16,), n0 >> 7, jnp.int32)
            col0 = jnp.full((16,), n0 & 127, jnp.int32)
            for c in range(4):
                vsp = plsc.load_gather(idx_v.at[p], [rowv, col0 + c])
                srow = vsp >> 2
                e0 = ((vsp << 5) & c127) + iota
                g0 = plsc.load_gather(table_v, [srow, e0])
                g1 = plsc.load_gather(table_v, [srow, e0 + 16])
                emb_v[p, g, pl.ds(c * 32, 16)] = g0
                emb_v[p, g, pl.ds(c * 32 + 16, 16)] = g1
            return carry2

        lax.fori_loop(0, _N_TILES // 4, grp_body, 0)
        pltpu.async_copy(emb_v.at[p], out_hbm.at[br], sem_out)

        @pl.when(i + 1 < _BPW)
        def _():
            pltpu.make_async_copy(board_hbm.at[br], idx_v.at[1 - p],
                                  sem_idx).wait()
        return carry

    lax.fori_loop(0, _BPW, row_body, 0)
    pltpu.make_async_copy(emb_v.at[0], out_hbm.at[base], sem_out).wait()
    pltpu.make_async_copy(emb_v.at[0], out_hbm.at[base], sem_out).wait()


def _sc_gather(emb_table2, board3):
    mesh = plsc.VectorSubcoreMesh(core_axis_name="c", subcore_axis_name="s")
    f = pl.kernel(
        _sc_gather_body,
        out_type=jax.ShapeDtypeStruct((_B, _N_TILES * _EMBED // 128, 128),
                                      jnp.float32),
        mesh=mesh,
        scratch_types=[
            pltpu.VMEM((_N_TILES * _EMBED // 128, 128), jnp.float32),
            pltpu.VMEM((2, 8, 128), jnp.int32),
            pltpu.VMEM((2, _N_TILES * _EMBED // 128, 128), jnp.float32),
            pltpu.SemaphoreType.DMA,
            pltpu.SemaphoreType.DMA,
        ],
        compiler_params=pltpu.CompilerParams(use_tc_tiling_on_sc=True,
                                             needs_layout_passes=False),
    )
    return f(emb_table2, board3)


_BB = 512
_KB = 4096
_K = _N_TILES * _EMBED


def _mlp_body(flat_ref, w1_ref, b1_ref, w2_ref, b2_ref, w3_ref, b3_ref,
              out_ref, acc_ref):
    k = pl.program_id(0)
    b = pl.program_id(1)
    nk = pl.num_programs(0)
    x = flat_ref[...].reshape(_BB, _KB)
    part = jnp.dot(x, w1_ref[...], preferred_element_type=jnp.float32)
    sl = pl.ds(b * _BB, _BB)

    @pl.when(k == 0)
    def _():
        acc_ref[sl, :] = part

    @pl.when(k > 0)
    def _():
        acc_ref[sl, :] = acc_ref[sl, :] + part

    @pl.when(k == nk - 1)
    def _():
        h1 = jnp.maximum(acc_ref[sl, :] + b1_ref[...], 0.0)
        h2 = jnp.dot(h1, w2_ref[...], preferred_element_type=jnp.float32)
        h2 = jnp.maximum(h2 + b2_ref[...], 0.0)
        out_ref[...] = (
            jnp.dot(h2, w3_ref[...], preferred_element_type=jnp.float32)
            + b3_ref[...]
        )


def _tc_mlp(flat3, W1, b1, W2, b2, W3, b3):
    grid = (_K // _KB, _B // _BB)
    return pl.pallas_call(
        _mlp_body,
        grid=grid,
        in_specs=[
            pl.BlockSpec((_BB, _KB // 128, 128), lambda k, b: (b, k, 0)),
            pl.BlockSpec((_KB, _HID), lambda k, b: (k, 0)),
            pl.BlockSpec((1, _HID), lambda k, b: (0, 0)),
            pl.BlockSpec((_HID, _HID), lambda k, b: (0, 0)),
            pl.BlockSpec((1, _HID), lambda k, b: (0, 0)),
            pl.BlockSpec((_HID, _NA), lambda k, b: (0, 0)),
            pl.BlockSpec((1, _NA), lambda k, b: (0, 0)),
        ],
        out_specs=pl.BlockSpec((_BB, _NA), lambda k, b: (b, 0)),
        out_shape=jax.ShapeDtypeStruct((_B, _NA), jnp.float32),
        scratch_shapes=[pltpu.VMEM((_B, _HID), jnp.float32)],
        compiler_params=pltpu.CompilerParams(
            dimension_semantics=("arbitrary", "arbitrary"),
        ),
    )(flat3, W1, b1, W2, b2, W3, b3)


def kernel(board, emb_table, W1, b1, W2, b2, W3, b3):
    board3 = board.astype(jnp.int32).reshape(_B, 8, 128)
    emb_table2 = emb_table.reshape(_N_TILES * _EMBED // 128, 128)
    flat3 = _sc_gather(emb_table2, board3)
    return _tc_mlp(
        flat3, W1, b1.reshape(1, _HID), W2, b2.reshape(1, _HID),
        W3, b3.reshape(1, _NA),
    )


# speedup vs baseline: 8.0039x; 3.7068x over previous
import jax
import jax.numpy as jnp
from jax import lax
from jax.experimental import pallas as pl
from jax.experimental.pallas import tpu as pltpu
from jax.experimental.pallas import tpu_sc as plsc

_N_TILES = 1024
_EMBED = 32
_HID = 256
_NA = 4
_B = 4096

_NC, _NS = 2, 16
_NW = _NC * _NS
_BPW = _B // _NW


def _sc_gather_body(table_hbm, board_hbm, out_hbm, table_v, idx_v, emb_v,
                    sem_out, sem_idx):
    w = lax.axis_index("s") * _NC + lax.axis_index("c")
    base = w * _BPW
    pltpu.sync_copy(table_hbm, table_v)
    iota = lax.iota(jnp.int32, 16)
    c127 = jnp.full((16,), 127, jnp.int32)

    pltpu.async_copy(board_hbm.at[base], idx_v.at[0], sem_idx).wait()

    def row_body(i, carry):
        br = base + i
        p = lax.rem(i, 2)
        @pl.when(i + 1 < _BPW)
        def _():
            pltpu.async_copy(board_hbm.at[br + 1], idx_v.at[1 - p], sem_idx)
        @pl.when(i >= 2)
        def _():
            pltpu.make_async_copy(emb_v.at[p], out_hbm.at[base], sem_out).wait()

        @plsc.parallel_loop(0, _N_TILES // 4, unroll=4)
        def grp_body(g):
            n0 = g * 4
            rowv = jnp.full((16,), n0 >> 7, jnp.int32)
            col0 = jnp.full((16,), n0 & 127, jnp.int32)
            for c in range(4):
                vsp = plsc.load_gather(idx_v.at[p], [rowv, col0 + c])
                srow = vsp >> 2
                e0 = ((vsp << 5) & c127) + iota
                g0 = plsc.load_gather(table_v, [srow, e0])
                g1 = plsc.load_gather(table_v, [srow, e0 + 16])
                emb_v[p, g, pl.ds(c * 32, 16)] = g0
                emb_v[p, g, pl.ds(c * 32 + 16, 16)] = g1
        pltpu.async_copy(emb_v.at[p], out_hbm.at[br], sem_out)

        @pl.when(i + 1 < _BPW)
        def _():
            pltpu.make_async_copy(board_hbm.at[br], idx_v.at[1 - p],
                                  sem_idx).wait()
        return carry

    lax.fori_loop(0, _BPW, row_body, 0)
    pltpu.make_async_copy(emb_v.at[0], out_hbm.at[base], sem_out).wait()
    pltpu.make_async_copy(emb_v.at[0], out_hbm.at[base], sem_out).wait()


def _sc_gather(emb_table2, board3):
    mesh = plsc.VectorSubcoreMesh(core_axis_name="c", subcore_axis_name="s")
    f = pl.kernel(
        _sc_gather_body,
        out_type=jax.ShapeDtypeStruct((_B, _N_TILES * _EMBED // 128, 128),
                                      jnp.float32),
        mesh=mesh,
        scratch_types=[
            pltpu.VMEM((_N_TILES * _EMBED // 128, 128), jnp.float32),
            pltpu.VMEM((2, 8, 128), jnp.int32),
            pltpu.VMEM((2, _N_TILES * _EMBED // 128, 128), jnp.float32),
            pltpu.SemaphoreType.DMA,
            pltpu.SemaphoreType.DMA,
        ],
        compiler_params=pltpu.CompilerParams(use_tc_tiling_on_sc=True,
                                             needs_layout_passes=False),
    )
    return f(emb_table2, board3)


_BB = 512
_KB = 4096
_K = _N_TILES * _EMBED


def _mlp_body(flat_ref, w1_ref, b1_ref, w2_ref, b2_ref, w3_ref, b3_ref,
              out_ref, acc_ref):
    k = pl.program_id(0)
    b = pl.program_id(1)
    nk = pl.num_programs(0)
    x = flat_ref[...].reshape(_BB, _KB)
    part = jnp.dot(x, w1_ref[...], preferred_element_type=jnp.float32)
    sl = pl.ds(b * _BB, _BB)

    @pl.when(k == 0)
    def _():
        acc_ref[sl, :] = part

    @pl.when(k > 0)
    def _():
        acc_ref[sl, :] = acc_ref[sl, :] + part

    @pl.when(k == nk - 1)
    def _():
        h1 = jnp.maximum(acc_ref[sl, :] + b1_ref[...], 0.0)
        h2 = jnp.dot(h1, w2_ref[...], preferred_element_type=jnp.float32)
        h2 = jnp.maximum(h2 + b2_ref[...], 0.0)
        out_ref[...] = (
            jnp.dot(h2, w3_ref[...], preferred_element_type=jnp.float32)
            + b3_ref[...]
        )


def _tc_mlp(flat3, W1, b1, W2, b2, W3, b3):
    grid = (_K // _KB, _B // _BB)
    return pl.pallas_call(
        _mlp_body,
        grid=grid,
        in_specs=[
            pl.BlockSpec((_BB, _KB // 128, 128), lambda k, b: (b, k, 0)),
            pl.BlockSpec((_KB, _HID), lambda k, b: (k, 0)),
            pl.BlockSpec((1, _HID), lambda k, b: (0, 0)),
            pl.BlockSpec((_HID, _HID), lambda k, b: (0, 0)),
            pl.BlockSpec((1, _HID), lambda k, b: (0, 0)),
            pl.BlockSpec((_HID, _NA), lambda k, b: (0, 0)),
            pl.BlockSpec((1, _NA), lambda k, b: (0, 0)),
        ],
        out_specs=pl.BlockSpec((_BB, _NA), lambda k, b: (b, 0)),
        out_shape=jax.ShapeDtypeStruct((_B, _NA), jnp.float32),
        scratch_shapes=[pltpu.VMEM((_B, _HID), jnp.float32)],
        compiler_params=pltpu.CompilerParams(
            dimension_semantics=("arbitrary", "arbitrary"),
        ),
    )(flat3, W1, b1, W2, b2, W3, b3)


def kernel(board, emb_table, W1, b1, W2, b2, W3, b3):
    board3 = board.astype(jnp.int32).reshape(_B, 8, 128)
    emb_table2 = emb_table.reshape(_N_TILES * _EMBED // 128, 128)
    flat3 = _sc_gather(emb_table2, board3)
    return _tc_mlp(
        flat3, W1, b1.reshape(1, _HID), W2, b2.reshape(1, _HID),
        W3, b3.reshape(1, _NA),
    )
